# initial kernel scaffold (unmeasured)
import jax
import jax.numpy as jnp
import numpy as np
from jax import lax
from jax.experimental import pallas as pl
from jax.experimental.pallas import tpu as pltpu

N_DEV = 4
B = 2
S_LOC = 512
D = 1024
HQ = 8
DH = 128
SCALE = 0.08838834764831843

_PERM = np.concatenate(
    [
        h * DH + np.concatenate([np.arange(0, DH, 2), np.arange(1, DH, 2)])
        for h in range(HQ)
    ]
)

_CompilerParams = getattr(pltpu, "CompilerParams", None) or getattr(
    pltpu, "TPUCompilerParams"
)


def kernel(x, Wq, Wk, Wv, Wo):
    my = lax.axis_index("i")

    pos = (my * S_LOC + jnp.arange(S_LOC))[:, None].astype(jnp.float32)
    inv = jnp.asarray(
        1.0 / (10000.0 ** (np.arange(0, DH, 2) / DH)), dtype=jnp.float32
    )
    ang = pos * inv[None, :]
    cos = jnp.concatenate([jnp.cos(ang), jnp.cos(ang)], axis=-1)
    sin = jnp.concatenate([jnp.sin(ang), jnp.sin(ang)], axis=-1)
    cos2 = jnp.tile(cos, (B, HQ))
    sin2 = jnp.tile(sin, (B, HQ))

    Wq_p = Wq[:, _PERM]
    Wk_p = Wk[:, _PERM]

    def body(
        x_ref,
        wq_ref,
        wk_ref,
        wv_ref,
        wo_ref,
        cos_ref,
        sin_ref,
        out_ref,
        k_all,
        v_all,
        ctx_ref,
        send_k,
        recv_k,
        send_v,
        recv_v,
    ):
        my_pos = lax.axis_index("i")
        left = (my_pos - 1) % N_DEV
        right = (my_pos + 1) % N_DEV

        barrier_sem = pltpu.get_barrier_semaphore()
        for nbr in (left, right):
            pl.semaphore_signal(
                barrier_sem,
                inc=1,
                device_id=(nbr,),
                device_id_type=pl.DeviceIdType.MESH,
            )
        pl.semaphore_wait(barrier_sem, 2)

        xb = x_ref[...].reshape(B * S_LOC, D).astype(jnp.bfloat16)
        cos2d = cos_ref[...]
        sin2d = sin_ref[...]

        def rope2d(t):
            halves = []
            for hd in range(HQ):
                lo = t[:, hd * DH : hd * DH + DH // 2]
                hi = t[:, hd * DH + DH // 2 : (hd + 1) * DH]
                halves += [-hi, lo]
            t_rot = jnp.concatenate(halves, axis=-1)
            return t * cos2d + t_rot * sin2d

        q = jnp.dot(
            xb, wq_ref[...].astype(jnp.bfloat16), preferred_element_type=jnp.float32
        )
        q = rope2d(q).astype(jnp.bfloat16)
        k = jnp.dot(
            xb, wk_ref[...].astype(jnp.bfloat16), preferred_element_type=jnp.float32
        )
        k = rope2d(k).astype(jnp.bfloat16)
        v = jnp.dot(
            xb, wv_ref[...].astype(jnp.bfloat16), preferred_element_type=jnp.float32
        ).astype(jnp.bfloat16)

        k_all[0, :, :] = k
        v_all[0, :, :] = v

        for h in range(N_DEV - 1):
            rk = pltpu.make_async_remote_copy(
                src_ref=k_all.at[h],
                dst_ref=k_all.at[h + 1],
                send_sem=send_k.at[h],
                recv_sem=recv_k.at[h],
                device_id=(right,),
                device_id_type=pl.DeviceIdType.MESH,
            )
            rv = pltpu.make_async_remote_copy(
                src_ref=v_all.at[h],
                dst_ref=v_all.at[h + 1],
                send_sem=send_v.at[h],
                recv_sem=recv_v.at[h],
                device_id=(right,),
                device_id_type=pl.DeviceIdType.MESH,
            )
            rk.start()
            rv.start()
            rk.wait()
            rv.wait()

        for b in range(B):
            for hd in range(HQ):
                rs = slice(b * S_LOC, (b + 1) * S_LOC)
                cs = slice(hd * DH, (hd + 1) * DH)
                qbh = q[rs, cs]
                kk = jnp.concatenate(
                    [k_all[r, rs, cs] for r in range(N_DEV)], axis=0
                )
                s = lax.dot_general(
                    qbh,
                    kk,
                    (((1,), (1,)), ((), ())),
                    preferred_element_type=jnp.float32,
                )
                s = s * SCALE
                m = jnp.max(s, axis=-1, keepdims=True)
                w = jnp.exp(s - m)
                w = (w / jnp.sum(w, axis=-1, keepdims=True)).astype(jnp.bfloat16)
                vv = jnp.concatenate(
                    [v_all[r, rs, cs] for r in range(N_DEV)], axis=0
                )
                ctx = jnp.dot(w, vv, preferred_element_type=jnp.float32)
                ctx_ref[rs, cs] = ctx.astype(jnp.bfloat16)

        out = jnp.dot(
            ctx_ref[...],
            wo_ref[...].astype(jnp.bfloat16),
            preferred_element_type=jnp.float32,
        )
        out_ref[...] = out.reshape(B, S_LOC, D)

    return pl.pallas_call(
        body,
        out_shape=jax.ShapeDtypeStruct((B, S_LOC, D), jnp.float32),
        in_specs=[pl.BlockSpec(memory_space=pltpu.VMEM)] * 7,
        out_specs=pl.BlockSpec(memory_space=pltpu.VMEM),
        scratch_shapes=[
            pltpu.VMEM((N_DEV, B * S_LOC, D), jnp.bfloat16),
            pltpu.VMEM((N_DEV, B * S_LOC, D), jnp.bfloat16),
            pltpu.VMEM((B * S_LOC, D), jnp.bfloat16),
            pltpu.SemaphoreType.DMA((N_DEV - 1,)),
            pltpu.SemaphoreType.DMA((N_DEV - 1,)),
            pltpu.SemaphoreType.DMA((N_DEV - 1,)),
            pltpu.SemaphoreType.DMA((N_DEV - 1,)),
        ],
        compiler_params=_CompilerParams(collective_id=0),
    )(x, Wq_p, Wk_p, Wv, Wo, cos2, sin2)


# baseline (device time: 237688 ns/iter reference)
import jax
import jax.numpy as jnp
import numpy as np
from jax import lax
from jax.experimental import pallas as pl
from jax.experimental.pallas import tpu as pltpu

N_DEV = 4
B = 2
S_LOC = 512
D = 1024
HQ = 8
DH = 128
SCALE = 0.08838834764831843

_PERM = np.concatenate(
    [
        h * DH + np.concatenate([np.arange(0, DH, 2), np.arange(1, DH, 2)])
        for h in range(HQ)
    ]
)

_CompilerParams = getattr(pltpu, "CompilerParams", None) or getattr(
    pltpu, "TPUCompilerParams"
)


def kernel(x, Wq, Wk, Wv, Wo):
    my = lax.axis_index("i")

    pos = (my * S_LOC + jnp.arange(S_LOC))[:, None].astype(jnp.float32)
    inv = jnp.asarray(
        1.0 / (10000.0 ** (np.arange(0, DH, 2) / DH)), dtype=jnp.float32
    )
    ang = pos * inv[None, :]
    cos = jnp.concatenate([jnp.cos(ang), jnp.cos(ang)], axis=-1)
    sin = jnp.concatenate([jnp.sin(ang), jnp.sin(ang)], axis=-1)
    cos2 = jnp.tile(cos, (B, HQ))
    sin2 = jnp.tile(sin, (B, HQ))

    Wq_p = Wq[:, _PERM].astype(jnp.bfloat16)
    Wk_p = Wk[:, _PERM].astype(jnp.bfloat16)
    Wv_c = Wv.astype(jnp.bfloat16)
    Wo_c = Wo.astype(jnp.bfloat16)
    x_c = x.astype(jnp.bfloat16)

    def body(
        x_ref,
        wq_ref,
        wk_ref,
        wv_ref,
        wo_ref,
        cos_ref,
        sin_ref,
        out_ref,
        k_all,
        v_all,
        ctx_ref,
        send_k,
        recv_k,
        send_v,
        recv_v,
    ):
        my_pos = lax.axis_index("i")
        left = (my_pos - 1) % N_DEV
        right = (my_pos + 1) % N_DEV

        barrier_sem = pltpu.get_barrier_semaphore()
        for nbr in (left, right):
            pl.semaphore_signal(
                barrier_sem,
                inc=1,
                device_id=(nbr,),
                device_id_type=pl.DeviceIdType.MESH,
            )
        pl.semaphore_wait(barrier_sem, 2)

        xb = x_ref[...].reshape(B * S_LOC, D)
        cos2d = cos_ref[...]
        sin2d = sin_ref[...]

        def rope2d(t):
            halves = []
            for hd in range(HQ):
                lo = t[:, hd * DH : hd * DH + DH // 2]
                hi = t[:, hd * DH + DH // 2 : (hd + 1) * DH]
                halves += [-hi, lo]
            t_rot = jnp.concatenate(halves, axis=-1)
            return t * cos2d + t_rot * sin2d

        q = jnp.dot(xb, wq_ref[...], preferred_element_type=jnp.float32)
        q = rope2d(q).astype(jnp.bfloat16)
        k = jnp.dot(xb, wk_ref[...], preferred_element_type=jnp.float32)
        k = rope2d(k).astype(jnp.bfloat16)
        v = jnp.dot(xb, wv_ref[...], preferred_element_type=jnp.float32).astype(
            jnp.bfloat16
        )

        k_all[0, :, :] = k
        v_all[0, :, :] = v

        for h in range(N_DEV - 1):
            rk = pltpu.make_async_remote_copy(
                src_ref=k_all.at[h],
                dst_ref=k_all.at[h + 1],
                send_sem=send_k.at[h],
                recv_sem=recv_k.at[h],
                device_id=(right,),
                device_id_type=pl.DeviceIdType.MESH,
            )
            rv = pltpu.make_async_remote_copy(
                src_ref=v_all.at[h],
                dst_ref=v_all.at[h + 1],
                send_sem=send_v.at[h],
                recv_sem=recv_v.at[h],
                device_id=(right,),
                device_id_type=pl.DeviceIdType.MESH,
            )
            rk.start()
            rv.start()
            rk.wait()
            rv.wait()

        for b in range(B):
            for hd in range(HQ):
                rs = slice(b * S_LOC, (b + 1) * S_LOC)
                cs = slice(hd * DH, (hd + 1) * DH)
                qbh = q[rs, cs]
                kk = jnp.concatenate(
                    [k_all[r, rs, cs] for r in range(N_DEV)], axis=0
                )
                s = lax.dot_general(
                    qbh,
                    kk,
                    (((1,), (1,)), ((), ())),
                    preferred_element_type=jnp.float32,
                )
                s = s * SCALE
                m = jnp.max(s, axis=-1, keepdims=True)
                w = jnp.exp(s - m)
                w = (w / jnp.sum(w, axis=-1, keepdims=True)).astype(jnp.bfloat16)
                vv = jnp.concatenate(
                    [v_all[r, rs, cs] for r in range(N_DEV)], axis=0
                )
                ctx = jnp.dot(w, vv, preferred_element_type=jnp.float32)
                ctx_ref[rs, cs] = ctx.astype(jnp.bfloat16)

        out = jnp.dot(ctx_ref[...], wo_ref[...], preferred_element_type=jnp.float32)
        out_ref[...] = out.reshape(B, S_LOC, D)

    return pl.pallas_call(
        body,
        out_shape=jax.ShapeDtypeStruct((B, S_LOC, D), jnp.float32),
        in_specs=[pl.BlockSpec(memory_space=pltpu.VMEM)] * 7,
        out_specs=pl.BlockSpec(memory_space=pltpu.VMEM),
        scratch_shapes=[
            pltpu.VMEM((N_DEV, B * S_LOC, D), jnp.bfloat16),
            pltpu.VMEM((N_DEV, B * S_LOC, D), jnp.bfloat16),
            pltpu.VMEM((B * S_LOC, D), jnp.bfloat16),
            pltpu.SemaphoreType.DMA((N_DEV - 1,)),
            pltpu.SemaphoreType.DMA((N_DEV - 1,)),
            pltpu.SemaphoreType.DMA((N_DEV - 1,)),
            pltpu.SemaphoreType.DMA((N_DEV - 1,)),
        ],
        compiler_params=_CompilerParams(
            collective_id=0, vmem_limit_bytes=100 * 1024 * 1024
        ),
    )(x_c, Wq_p, Wk_p, Wv_c, Wo_c, cos2, sin2)


# device time: 134500 ns/iter; 1.7672x vs baseline; 1.7672x over previous
import jax
import jax.numpy as jnp
import numpy as np
from jax import lax
from jax.experimental import pallas as pl
from jax.experimental.pallas import tpu as pltpu

N_DEV = 4
B = 2
S_LOC = 512
D = 1024
HQ = 8
DH = 128
HALF = D // 2
SCALE = 0.08838834764831843

_PERM = np.concatenate(
    [
        h * DH + np.concatenate([np.arange(0, DH, 2), np.arange(1, DH, 2)])
        for h in range(HQ)
    ]
)

_CompilerParams = getattr(pltpu, "CompilerParams", None) or getattr(
    pltpu, "TPUCompilerParams"
)


def kernel(x, Wq, Wk, Wv, Wo):
    my = lax.axis_index("i")

    pos = (my * S_LOC + jnp.arange(S_LOC))[:, None].astype(jnp.float32)
    inv = jnp.asarray(
        1.0 / (10000.0 ** (np.arange(0, DH, 2) / DH)), dtype=jnp.float32
    )
    ang = pos * inv[None, :]
    cos = jnp.concatenate([jnp.cos(ang), jnp.cos(ang)], axis=-1)
    sin = jnp.concatenate([jnp.sin(ang), jnp.sin(ang)], axis=-1)

    Wq_p = Wq[:, _PERM].astype(jnp.bfloat16)
    Wk_p = Wk[:, _PERM].astype(jnp.bfloat16)
    Wv_c = Wv.astype(jnp.bfloat16)
    Wo_c = Wo.astype(jnp.bfloat16)
    x_c = x.astype(jnp.bfloat16)

    def body(
        x_ref,
        wq_ref,
        wk_ref,
        wv_ref,
        wo_ref,
        cos_ref,
        sin_ref,
        out_ref,
        rbuf,
        lbuf,
        ctx_ref,
        acc_ref,
        r_send,
        r_recv,
        l_send,
        l_recv,
    ):
        my_pos = lax.axis_index("i")
        left = (my_pos - 1) % N_DEV
        right = (my_pos + 1) % N_DEV

        barrier_sem = pltpu.get_barrier_semaphore()
        for nbr in (left, right):
            pl.semaphore_signal(
                barrier_sem,
                inc=1,
                device_id=(nbr,),
                device_id_type=pl.DeviceIdType.MESH,
            )
        pl.semaphore_wait(barrier_sem, 2)

        xb = x_ref[...].reshape(B * S_LOC, D)
        cos_t = jnp.concatenate([cos_ref[...], cos_ref[...]], axis=0)
        sin_t = jnp.concatenate([sin_ref[...], sin_ref[...]], axis=0)

        def rope2d(t):
            blocks = []
            for hd in range(HQ):
                tb = t[:, hd * DH : (hd + 1) * DH]
                tb_rot = jnp.concatenate(
                    [-tb[:, DH // 2 :], tb[:, : DH // 2]], axis=-1
                )
                blocks.append(tb * cos_t + tb_rot * sin_t)
            return jnp.concatenate(blocks, axis=-1)

        k = jnp.dot(xb, wk_ref[...], preferred_element_type=jnp.float32)
        k = rope2d(k).astype(jnp.bfloat16)
        v = jnp.dot(xb, wv_ref[...], preferred_element_type=jnp.float32).astype(
            jnp.bfloat16
        )
        rbuf[0, :, :HALF] = k[:, :HALF]
        rbuf[0, :, HALF:] = v[:, :HALF]
        lbuf[0, :, :HALF] = k[:, HALF:]
        lbuf[0, :, HALF:] = v[:, HALF:]

        def make_hop(h):
            r = pltpu.make_async_remote_copy(
                src_ref=rbuf.at[h],
                dst_ref=rbuf.at[h + 1],
                send_sem=r_send.at[h],
                recv_sem=r_recv.at[h],
                device_id=(right,),
                device_id_type=pl.DeviceIdType.MESH,
            )
            l = pltpu.make_async_remote_copy(
                src_ref=lbuf.at[h],
                dst_ref=lbuf.at[h + 1],
                send_sem=l_send.at[h],
                recv_sem=l_recv.at[h],
                device_id=(left,),
                device_id_type=pl.DeviceIdType.MESH,
            )
            return r, l

        hop0 = make_hop(0)
        hop0[0].start()
        hop0[1].start()

        q = jnp.dot(xb, wq_ref[...], preferred_element_type=jnp.float32)
        q = rope2d(q).astype(jnp.bfloat16)

        m_st = [[None] * HQ for _ in range(B)]
        l_st = [[None] * HQ for _ in range(B)]

        def fold(b, hd, kbh, vbh, first):
            rs = slice(b * S_LOC, (b + 1) * S_LOC)
            cs = slice(hd * DH, (hd + 1) * DH)
            qbh = q[rs, cs]
            s = (
                lax.dot_general(
                    qbh,
                    kbh,
                    (((1,), (1,)), ((), ())),
                    preferred_element_type=jnp.float32,
                )
                * SCALE
            )
            m_new = jnp.max(s, axis=-1, keepdims=True)
            if not first:
                m_new = jnp.maximum(m_st[b][hd], m_new)
            p = jnp.exp(s - m_new)
            pv = jnp.dot(
                p.astype(jnp.bfloat16), vbh, preferred_element_type=jnp.float32
            )
            if first:
                l_st[b][hd] = jnp.sum(p, axis=-1, keepdims=True)
                acc_ref[rs, cs] = pv
            else:
                corr = jnp.exp(m_st[b][hd] - m_new)
                l_st[b][hd] = l_st[b][hd] * corr + jnp.sum(
                    p, axis=-1, keepdims=True
                )
                acc_ref[rs, cs] = acc_ref[rs, cs] * corr + pv
            m_st[b][hd] = m_new

        def fold_slot(slot_k, slot_v, heads, first=False):
            for b in range(B):
                rs = slice(b * S_LOC, (b + 1) * S_LOC)
                for j, hd in enumerate(heads):
                    cs = slice(j * DH, (j + 1) * DH)
                    fold(b, hd, slot_k[rs, cs], slot_v[rs, cs], first)

        r0 = rbuf[0]
        l0 = lbuf[0]
        fold_slot(r0[:, :HALF], r0[:, HALF:], (0, 1, 2, 3), first=True)
        fold_slot(l0[:, :HALF], l0[:, HALF:], (4, 5, 6, 7), first=True)

        hops = {0: hop0}
        for h in range(N_DEV - 1):
            hops[h][0].wait()
            hops[h][1].wait()
            if h + 1 < N_DEV - 1:
                nxt = make_hop(h + 1)
                nxt[0].start()
                nxt[1].start()
                hops[h + 1] = nxt
            rslot = rbuf[h + 1]
            lslot = lbuf[h + 1]
            fold_slot(rslot[:, :HALF], rslot[:, HALF:], (0, 1, 2, 3))
            fold_slot(lslot[:, :HALF], lslot[:, HALF:], (4, 5, 6, 7))

        for b in range(B):
            rs = slice(b * S_LOC, (b + 1) * S_LOC)
            for hd in range(HQ):
                cs = slice(hd * DH, (hd + 1) * DH)
                ctx = acc_ref[rs, cs] / l_st[b][hd]
                ctx_ref[rs, cs] = ctx.astype(jnp.bfloat16)

        out = jnp.dot(ctx_ref[...], wo_ref[...], preferred_element_type=jnp.float32)
        out_ref[...] = out.reshape(B, S_LOC, D)

    return pl.pallas_call(
        body,
        out_shape=jax.ShapeDtypeStruct((B, S_LOC, D), jnp.float32),
        in_specs=[pl.BlockSpec(memory_space=pltpu.VMEM)] * 7,
        out_specs=pl.BlockSpec(memory_space=pltpu.VMEM),
        scratch_shapes=[
            pltpu.VMEM((N_DEV, B * S_LOC, D), jnp.bfloat16),
            pltpu.VMEM((N_DEV, B * S_LOC, D), jnp.bfloat16),
            pltpu.VMEM((B * S_LOC, D), jnp.bfloat16),
            pltpu.VMEM((B * S_LOC, D), jnp.float32),
            pltpu.SemaphoreType.DMA((N_DEV - 1,)),
            pltpu.SemaphoreType.DMA((N_DEV - 1,)),
            pltpu.SemaphoreType.DMA((N_DEV - 1,)),
            pltpu.SemaphoreType.DMA((N_DEV - 1,)),
        ],
        compiler_params=_CompilerParams(
            collective_id=0, vmem_limit_bytes=100 * 1024 * 1024
        ),
    )(x_c, Wq_p, Wk_p, Wv_c, Wo_c, cos, sin)


# device time: 125735 ns/iter; 1.8904x vs baseline; 1.0697x over previous
import jax
import jax.numpy as jnp
import numpy as np
from jax import lax
from jax.experimental import pallas as pl
from jax.experimental.pallas import tpu as pltpu

N_DEV = 4
B = 2
S_LOC = 512
D = 1024
HQ = 8
DH = 128
HALF = D // 2
SCALE = 0.08838834764831843

_PERM = np.concatenate(
    [
        h * DH + np.concatenate([np.arange(0, DH, 2), np.arange(1, DH, 2)])
        for h in range(HQ)
    ]
)

_CompilerParams = getattr(pltpu, "CompilerParams", None) or getattr(
    pltpu, "TPUCompilerParams"
)


def kernel(x, Wq, Wk, Wv, Wo):
    my = lax.axis_index("i")

    pos = (my * S_LOC + jnp.arange(S_LOC))[:, None].astype(jnp.float32)
    inv = jnp.asarray(
        1.0 / (10000.0 ** (np.arange(0, DH, 2) / DH)), dtype=jnp.float32
    )
    ang = pos * inv[None, :]
    cos = jnp.concatenate([jnp.cos(ang), jnp.cos(ang)], axis=-1).astype(
        jnp.bfloat16
    )
    sin = jnp.concatenate([jnp.sin(ang), jnp.sin(ang)], axis=-1).astype(
        jnp.bfloat16
    )

    Wq_p = Wq[:, _PERM].astype(jnp.bfloat16)
    Wk_p = Wk[:, _PERM].astype(jnp.bfloat16)
    Wv_c = Wv.astype(jnp.bfloat16)
    Wo_c = Wo.astype(jnp.bfloat16)
    x_c = x.astype(jnp.bfloat16)

    def body(
        x_ref,
        wq_ref,
        wk_ref,
        wv_ref,
        wo_ref,
        cos_ref,
        sin_ref,
        out_ref,
        rbuf,
        lbuf,
        ctx_ref,
        acc_ref,
        r_send,
        r_recv,
        l_send,
        l_recv,
    ):
        my_pos = lax.axis_index("i")
        left = (my_pos - 1) % N_DEV
        right = (my_pos + 1) % N_DEV

        barrier_sem = pltpu.get_barrier_semaphore()
        for nbr in (left, right):
            pl.semaphore_signal(
                barrier_sem,
                inc=1,
                device_id=(nbr,),
                device_id_type=pl.DeviceIdType.MESH,
            )
        pl.semaphore_wait(barrier_sem, 2)

        xb = x_ref[...].reshape(B * S_LOC, D)
        cos_t = jnp.concatenate([cos_ref[...], cos_ref[...]], axis=0)
        sin_t = jnp.concatenate([sin_ref[...], sin_ref[...]], axis=0)

        def rope2d(t):
            blocks = []
            for hd in range(HQ):
                tb = t[:, hd * DH : (hd + 1) * DH]
                tb_rot = jnp.concatenate(
                    [-tb[:, DH // 2 :], tb[:, : DH // 2]], axis=-1
                )
                blocks.append(tb * cos_t + tb_rot * sin_t)
            return jnp.concatenate(blocks, axis=-1)

        k = jnp.dot(xb, wk_ref[...], preferred_element_type=jnp.float32)
        k = rope2d(k.astype(jnp.bfloat16))
        v = jnp.dot(xb, wv_ref[...], preferred_element_type=jnp.float32).astype(
            jnp.bfloat16
        )
        rbuf[0, :, :HALF] = k[:, :HALF]
        rbuf[0, :, HALF:] = v[:, :HALF]
        lbuf[0, :, :HALF] = k[:, HALF:]
        lbuf[0, :, HALF:] = v[:, HALF:]

        started = []

        def rows(s):
            return pl.ds(s * S_LOC, S_LOC)

        def start_hop(h, s):
            r = pltpu.make_async_remote_copy(
                src_ref=rbuf.at[h, rows(s)],
                dst_ref=rbuf.at[h + 1, rows(s)],
                send_sem=r_send.at[h, s],
                recv_sem=r_recv.at[h, s],
                device_id=(right,),
                device_id_type=pl.DeviceIdType.MESH,
            )
            l = pltpu.make_async_remote_copy(
                src_ref=lbuf.at[h, rows(s)],
                dst_ref=lbuf.at[h + 1, rows(s)],
                send_sem=l_send.at[h, s],
                recv_sem=l_recv.at[h, s],
                device_id=(left,),
                device_id_type=pl.DeviceIdType.MESH,
            )
            r.start()
            l.start()
            started.append(r)
            started.append(l)
            return r, l

        def wait_arrival(pair):
            pair[0].wait_recv()
            pair[1].wait_recv()

        hop0 = [start_hop(0, 0), start_hop(0, 1)]

        q = jnp.dot(xb, wq_ref[...], preferred_element_type=jnp.float32)
        q = rope2d(q.astype(jnp.bfloat16))

        m_st = [[None] * HQ for _ in range(B)]
        l_st = [[None] * HQ for _ in range(B)]

        def fold(b, hd, kbh, vbh, first):
            rs = slice(b * S_LOC, (b + 1) * S_LOC)
            cs = slice(hd * DH, (hd + 1) * DH)
            qbh = q[rs, cs]
            s = (
                lax.dot_general(
                    qbh,
                    kbh,
                    (((1,), (1,)), ((), ())),
                    preferred_element_type=jnp.float32,
                )
                * SCALE
            )
            m_new = jnp.max(s, axis=-1, keepdims=True)
            if not first:
                m_new = jnp.maximum(m_st[b][hd], m_new)
            p = jnp.exp(s - m_new)
            pv = jnp.dot(
                p.astype(jnp.bfloat16), vbh, preferred_element_type=jnp.float32
            )
            if first:
                l_st[b][hd] = jnp.sum(p, axis=-1, keepdims=True)
                acc_ref[rs, cs] = pv
            else:
                corr = jnp.exp(m_st[b][hd] - m_new)
                l_st[b][hd] = l_st[b][hd] * corr + jnp.sum(
                    p, axis=-1, keepdims=True
                )
                acc_ref[rs, cs] = acc_ref[rs, cs] * corr + pv
            m_st[b][hd] = m_new

        def fold_sub(slot, b, first=False):
            rsl = rbuf[slot, b * S_LOC : (b + 1) * S_LOC, :]
            lsl = lbuf[slot, b * S_LOC : (b + 1) * S_LOC, :]
            for j in range(HQ // 2):
                cs = slice(j * DH, (j + 1) * DH)
                vs = slice(HALF + j * DH, HALF + (j + 1) * DH)
                fold(b, j, rsl[:, cs], rsl[:, vs], first)
                fold(b, 4 + j, lsl[:, cs], lsl[:, vs], first)

        def finish_batch(b):
            rs = slice(b * S_LOC, (b + 1) * S_LOC)
            for hd in range(HQ):
                cs = slice(hd * DH, (hd + 1) * DH)
                ctx_ref[rs, cs] = (acc_ref[rs, cs] / l_st[b][hd]).astype(
                    jnp.bfloat16
                )
            out_b = jnp.dot(
                ctx_ref[rs, :], wo_ref[...], preferred_element_type=jnp.float32
            )
            out_ref[b, :, :] = out_b

        fold_sub(0, 0, first=True)
        fold_sub(0, 1, first=True)

        hops = {0: hop0}
        for h in range(N_DEV - 1):
            for s in range(B):
                wait_arrival(hops[h][s])
                if h + 1 < N_DEV - 1:
                    nxt = start_hop(h + 1, s)
                    hops.setdefault(h + 1, [None, None])[s] = nxt
                fold_sub(h + 1, s)
                if h == N_DEV - 2:
                    finish_batch(s)

        for rdma in started:
            rdma.wait_send()

    return pl.pallas_call(
        body,
        out_shape=jax.ShapeDtypeStruct((B, S_LOC, D), jnp.float32),
        in_specs=[pl.BlockSpec(memory_space=pltpu.VMEM)] * 7,
        out_specs=pl.BlockSpec(memory_space=pltpu.VMEM),
        scratch_shapes=[
            pltpu.VMEM((N_DEV, B * S_LOC, D), jnp.bfloat16),
            pltpu.VMEM((N_DEV, B * S_LOC, D), jnp.bfloat16),
            pltpu.VMEM((B * S_LOC, D), jnp.bfloat16),
            pltpu.VMEM((B * S_LOC, D), jnp.float32),
            pltpu.SemaphoreType.DMA((N_DEV - 1, B)),
            pltpu.SemaphoreType.DMA((N_DEV - 1, B)),
            pltpu.SemaphoreType.DMA((N_DEV - 1, B)),
            pltpu.SemaphoreType.DMA((N_DEV - 1, B)),
        ],
        compiler_params=_CompilerParams(
            collective_id=0, vmem_limit_bytes=100 * 1024 * 1024
        ),
    )(x_c, Wq_p, Wk_p, Wv_c, Wo_c, cos, sin)


# device time: 123281 ns/iter; 1.9280x vs baseline; 1.0199x over previous
import jax
import jax.numpy as jnp
import numpy as np
from jax import lax
from jax.experimental import pallas as pl
from jax.experimental.pallas import tpu as pltpu

N_DEV = 4
B = 2
S_LOC = 512
D = 1024
HQ = 8
DH = 128
HALF = D // 2
SCALE = 0.08838834764831843

_PERM = np.concatenate(
    [
        h * DH + np.concatenate([np.arange(0, DH, 2), np.arange(1, DH, 2)])
        for h in range(HQ)
    ]
)

_CompilerParams = getattr(pltpu, "CompilerParams", None) or getattr(
    pltpu, "TPUCompilerParams"
)


def kernel(x, Wq, Wk, Wv, Wo):
    my = lax.axis_index("i")

    pos = (my * S_LOC + jnp.arange(S_LOC))[:, None].astype(jnp.float32)
    inv = jnp.asarray(
        1.0 / (10000.0 ** (np.arange(0, DH, 2) / DH)), dtype=jnp.float32
    )
    ang = pos * inv[None, :]
    cos = jnp.concatenate([jnp.cos(ang), jnp.cos(ang)], axis=-1).astype(
        jnp.bfloat16
    )
    sin = jnp.concatenate([jnp.sin(ang), jnp.sin(ang)], axis=-1).astype(
        jnp.bfloat16
    )

    Wq_p = Wq[:, _PERM].astype(jnp.bfloat16)
    Wk_p = Wk[:, _PERM].astype(jnp.bfloat16)
    Wv_c = Wv.astype(jnp.bfloat16)
    Wo_c = Wo.astype(jnp.bfloat16)
    x_c = x.astype(jnp.bfloat16)

    def body(
        x_ref,
        wq_ref,
        wk_ref,
        wv_ref,
        wo_ref,
        cos_ref,
        sin_ref,
        out_ref,
        rbuf,
        lbuf,
        ctx_ref,
        acc_ref,
        r_send,
        r_recv,
        l_send,
        l_recv,
    ):
        my_pos = lax.axis_index("i")
        left = (my_pos - 1) % N_DEV
        right = (my_pos + 1) % N_DEV

        barrier_sem = pltpu.get_barrier_semaphore()
        for nbr in (left, right):
            pl.semaphore_signal(
                barrier_sem,
                inc=1,
                device_id=(nbr,),
                device_id_type=pl.DeviceIdType.MESH,
            )
        pl.semaphore_wait(barrier_sem, 2)

        cos_b = cos_ref[...]
        sin_b = sin_ref[...]

        def rope_batch(t):
            blocks = []
            for hd in range(HQ):
                tb = t[:, hd * DH : (hd + 1) * DH]
                tb_rot = jnp.concatenate(
                    [-tb[:, DH // 2 :], tb[:, : DH // 2]], axis=-1
                )
                blocks.append(tb * cos_b + tb_rot * sin_b)
            return jnp.concatenate(blocks, axis=-1)

        def proj_batch(b, w_ref, rope):
            xbb = x_ref[b, :, :]
            t = jnp.dot(xbb, w_ref[...], preferred_element_type=jnp.float32)
            t = t.astype(jnp.bfloat16)
            return rope_batch(t) if rope else t

        started = []

        def rows(s):
            return pl.ds(s * S_LOC, S_LOC)

        def start_hop(h, s):
            r = pltpu.make_async_remote_copy(
                src_ref=rbuf.at[h, rows(s)],
                dst_ref=rbuf.at[h + 1, rows(s)],
                send_sem=r_send.at[h, s],
                recv_sem=r_recv.at[h, s],
                device_id=(right,),
                device_id_type=pl.DeviceIdType.MESH,
            )
            l = pltpu.make_async_remote_copy(
                src_ref=lbuf.at[h, rows(s)],
                dst_ref=lbuf.at[h + 1, rows(s)],
                send_sem=l_send.at[h, s],
                recv_sem=l_recv.at[h, s],
                device_id=(left,),
                device_id_type=pl.DeviceIdType.MESH,
            )
            r.start()
            l.start()
            started.append(r)
            started.append(l)
            return r, l

        def wait_arrival(pair):
            pair[0].wait_recv()
            pair[1].wait_recv()

        hops = {0: [None, None]}
        for s in range(B):
            kb = proj_batch(s, wk_ref, rope=True)
            vb = proj_batch(s, wv_ref, rope=False)
            rbuf[0, rows(s), :HALF] = kb[:, :HALF]
            rbuf[0, rows(s), HALF:] = vb[:, :HALF]
            lbuf[0, rows(s), :HALF] = kb[:, HALF:]
            lbuf[0, rows(s), HALF:] = vb[:, HALF:]
            hops[0][s] = start_hop(0, s)

        q_b = [proj_batch(b, wq_ref, rope=True) for b in range(B)]

        m_st = [[None] * HQ for _ in range(B)]
        l_st = [[None] * HQ for _ in range(B)]

        def fold(b, hd, kbh, vbh, first):
            rs = slice(b * S_LOC, (b + 1) * S_LOC)
            cs = slice(hd * DH, (hd + 1) * DH)
            qbh = q_b[b][:, cs]
            s = (
                lax.dot_general(
                    qbh,
                    kbh,
                    (((1,), (1,)), ((), ())),
                    preferred_element_type=jnp.float32,
                )
                * SCALE
            )
            m_new = jnp.max(s, axis=-1, keepdims=True)
            if not first:
                m_new = jnp.maximum(m_st[b][hd], m_new)
            p = jnp.exp(s - m_new)
            pv = jnp.dot(
                p.astype(jnp.bfloat16), vbh, preferred_element_type=jnp.float32
            )
            if first:
                l_st[b][hd] = jnp.sum(p, axis=-1, keepdims=True)
                acc_ref[rs, cs] = pv
            else:
                corr = jnp.exp(m_st[b][hd] - m_new)
                l_st[b][hd] = l_st[b][hd] * corr + jnp.sum(
                    p, axis=-1, keepdims=True
                )
                acc_ref[rs, cs] = acc_ref[rs, cs] * corr + pv
            m_st[b][hd] = m_new

        def fold_sub(slot, b, first=False):
            rsl = rbuf[slot, b * S_LOC : (b + 1) * S_LOC, :]
            lsl = lbuf[slot, b * S_LOC : (b + 1) * S_LOC, :]
            for j in range(HQ // 2):
                cs = slice(j * DH, (j + 1) * DH)
                vs = slice(HALF + j * DH, HALF + (j + 1) * DH)
                fold(b, j, rsl[:, cs], rsl[:, vs], first)
                fold(b, 4 + j, lsl[:, cs], lsl[:, vs], first)

        def finish_batch(b):
            rs = slice(b * S_LOC, (b + 1) * S_LOC)
            for hd in range(HQ):
                cs = slice(hd * DH, (hd + 1) * DH)
                ctx_ref[rs, cs] = (acc_ref[rs, cs] / l_st[b][hd]).astype(
                    jnp.bfloat16
                )
            out_b = jnp.dot(
                ctx_ref[rs, :], wo_ref[...], preferred_element_type=jnp.float32
            )
            out_ref[b, :, :] = out_b

        fold_sub(0, 0, first=True)
        wait_arrival(hops[0][0])
        hops[1] = [start_hop(1, 0), None]
        fold_sub(0, 1, first=True)
        wait_arrival(hops[0][1])
        hops[1][1] = start_hop(1, 1)
        fold_sub(1, 0)
        wait_arrival(hops[1][0])
        hops[2] = [start_hop(2, 0), None]
        fold_sub(1, 1)
        wait_arrival(hops[1][1])
        hops[2][1] = start_hop(2, 1)
        fold_sub(2, 0)
        wait_arrival(hops[2][0])
        fold_sub(2, 1)
        fold_sub(3, 0)
        finish_batch(0)
        wait_arrival(hops[2][1])
        fold_sub(3, 1)
        finish_batch(1)

        for rdma in started:
            rdma.wait_send()

    return pl.pallas_call(
        body,
        out_shape=jax.ShapeDtypeStruct((B, S_LOC, D), jnp.float32),
        in_specs=[pl.BlockSpec(memory_space=pltpu.VMEM)] * 7,
        out_specs=pl.BlockSpec(memory_space=pltpu.VMEM),
        scratch_shapes=[
            pltpu.VMEM((N_DEV, B * S_LOC, D), jnp.bfloat16),
            pltpu.VMEM((N_DEV, B * S_LOC, D), jnp.bfloat16),
            pltpu.VMEM((B * S_LOC, D), jnp.bfloat16),
            pltpu.VMEM((B * S_LOC, D), jnp.float32),
            pltpu.SemaphoreType.DMA((N_DEV - 1, B)),
            pltpu.SemaphoreType.DMA((N_DEV - 1, B)),
            pltpu.SemaphoreType.DMA((N_DEV - 1, B)),
            pltpu.SemaphoreType.DMA((N_DEV - 1, B)),
        ],
        compiler_params=_CompilerParams(
            collective_id=0, vmem_limit_bytes=100 * 1024 * 1024
        ),
    )(x_c, Wq_p, Wk_p, Wv_c, Wo_c, cos, sin)


# device time: 121710 ns/iter; 1.9529x vs baseline; 1.0129x over previous
import jax
import jax.numpy as jnp
import numpy as np
from jax import lax
from jax.experimental import pallas as pl
from jax.experimental.pallas import tpu as pltpu

N_DEV = 4
B = 2
S_LOC = 512
D = 1024
HQ = 8
DH = 128
HALF = D // 2
SCALE = 0.08838834764831843

_PERM = np.concatenate(
    [
        h * DH + np.concatenate([np.arange(0, DH, 2), np.arange(1, DH, 2)])
        for h in range(HQ)
    ]
)

_CompilerParams = getattr(pltpu, "CompilerParams", None) or getattr(
    pltpu, "TPUCompilerParams"
)


def kernel(x, Wq, Wk, Wv, Wo):
    my = lax.axis_index("i")

    pos = (my * S_LOC + jnp.arange(S_LOC))[:, None].astype(jnp.float32)
    inv = jnp.asarray(
        1.0 / (10000.0 ** (np.arange(0, DH, 2) / DH)), dtype=jnp.float32
    )
    ang = pos * inv[None, :]
    cos = jnp.concatenate([jnp.cos(ang), jnp.cos(ang)], axis=-1).astype(
        jnp.bfloat16
    )
    sin = jnp.concatenate([jnp.sin(ang), jnp.sin(ang)], axis=-1).astype(
        jnp.bfloat16
    )

    Wq_p = Wq[:, _PERM].astype(jnp.bfloat16)
    Wk_p = Wk[:, _PERM].astype(jnp.bfloat16)
    Wv_c = Wv.astype(jnp.bfloat16)
    Wo_c = Wo.astype(jnp.bfloat16)
    x_c = x.astype(jnp.bfloat16)

    def body(
        x_ref,
        wq_ref,
        wk_ref,
        wv_ref,
        wo_ref,
        cos_ref,
        sin_ref,
        out_ref,
        rbuf,
        lbuf,
        ctx_ref,
        acc_ref,
        r_send,
        r_recv,
        l_send,
        l_recv,
    ):
        my_pos = lax.axis_index("i")
        left = (my_pos - 1) % N_DEV
        right = (my_pos + 1) % N_DEV

        barrier_sem = pltpu.get_barrier_semaphore()
        for nbr in (left, right):
            pl.semaphore_signal(
                barrier_sem,
                inc=1,
                device_id=(nbr,),
                device_id_type=pl.DeviceIdType.MESH,
            )
        pl.semaphore_wait(barrier_sem, 2)

        cos_b = cos_ref[...]
        sin_b = sin_ref[...]

        def rope_batch(t):
            blocks = []
            for hd in range(HQ):
                tb = t[:, hd * DH : (hd + 1) * DH]
                tb_rot = jnp.concatenate(
                    [-tb[:, DH // 2 :], tb[:, : DH // 2]], axis=-1
                )
                blocks.append(tb * cos_b + tb_rot * sin_b)
            return jnp.concatenate(blocks, axis=-1)

        def proj_batch(b, w_ref, rope):
            xbb = x_ref[b, :, :]
            t = jnp.dot(xbb, w_ref[...], preferred_element_type=jnp.float32)
            t = t.astype(jnp.bfloat16)
            return rope_batch(t) if rope else t

        started = []

        def rows(s):
            return pl.ds(s * S_LOC, S_LOC)

        def start_hop(h, s):
            r = pltpu.make_async_remote_copy(
                src_ref=rbuf.at[h, rows(s)],
                dst_ref=rbuf.at[h + 1, rows(s)],
                send_sem=r_send.at[h, s],
                recv_sem=r_recv.at[h, s],
                device_id=(right,),
                device_id_type=pl.DeviceIdType.MESH,
            )
            l = pltpu.make_async_remote_copy(
                src_ref=lbuf.at[h, rows(s)],
                dst_ref=lbuf.at[h + 1, rows(s)],
                send_sem=l_send.at[h, s],
                recv_sem=l_recv.at[h, s],
                device_id=(left,),
                device_id_type=pl.DeviceIdType.MESH,
            )
            r.start()
            l.start()
            started.append(r)
            started.append(l)
            return r, l

        def wait_arrival(pair):
            pair[0].wait_recv()
            pair[1].wait_recv()

        hops = {0: [None, None]}
        for s in range(B):
            kb = proj_batch(s, wk_ref, rope=True)
            vb = proj_batch(s, wv_ref, rope=False)
            rbuf[0, rows(s), :HALF] = kb[:, :HALF]
            rbuf[0, rows(s), HALF:] = vb[:, :HALF]
            lbuf[0, rows(s), :HALF] = kb[:, HALF:]
            lbuf[0, rows(s), HALF:] = vb[:, HALF:]
            hops[0][s] = start_hop(0, s)

        q_b = [
            (proj_batch(b, wq_ref, rope=True) * jnp.bfloat16(SCALE))
            for b in range(B)
        ]

        l_st = [[None] * HQ for _ in range(B)]

        def fold(b, hd, kbh, vbh, first):
            rs = slice(b * S_LOC, (b + 1) * S_LOC)
            cs = slice(hd * DH, (hd + 1) * DH)
            qbh = q_b[b][:, cs]
            s = lax.dot_general(
                qbh,
                kbh,
                (((1,), (1,)), ((), ())),
                preferred_element_type=jnp.float32,
            )
            p = jnp.exp(s)
            pv = jnp.dot(
                p.astype(jnp.bfloat16), vbh, preferred_element_type=jnp.float32
            )
            if first:
                l_st[b][hd] = jnp.sum(p, axis=-1, keepdims=True)
                acc_ref[rs, cs] = pv
            else:
                l_st[b][hd] = l_st[b][hd] + jnp.sum(p, axis=-1, keepdims=True)
                acc_ref[rs, cs] = acc_ref[rs, cs] + pv

        def fold_sub(slot, b, first=False):
            rsl = rbuf[slot, b * S_LOC : (b + 1) * S_LOC, :]
            lsl = lbuf[slot, b * S_LOC : (b + 1) * S_LOC, :]
            for j in range(HQ // 2):
                cs = slice(j * DH, (j + 1) * DH)
                vs = slice(HALF + j * DH, HALF + (j + 1) * DH)
                fold(b, j, rsl[:, cs], rsl[:, vs], first)
                fold(b, 4 + j, lsl[:, cs], lsl[:, vs], first)

        def finish_batch(b):
            rs = slice(b * S_LOC, (b + 1) * S_LOC)
            for hd in range(HQ):
                cs = slice(hd * DH, (hd + 1) * DH)
                ctx_ref[rs, cs] = (acc_ref[rs, cs] / l_st[b][hd]).astype(
                    jnp.bfloat16
                )
            out_b = jnp.dot(
                ctx_ref[rs, :], wo_ref[...], preferred_element_type=jnp.float32
            )
            out_ref[b, :, :] = out_b

        fold_sub(0, 0, first=True)
        wait_arrival(hops[0][0])
        hops[1] = [start_hop(1, 0), None]
        fold_sub(0, 1, first=True)
        wait_arrival(hops[0][1])
        hops[1][1] = start_hop(1, 1)
        fold_sub(1, 0)
        wait_arrival(hops[1][0])
        hops[2] = [start_hop(2, 0), None]
        fold_sub(1, 1)
        wait_arrival(hops[1][1])
        hops[2][1] = start_hop(2, 1)
        fold_sub(2, 0)
        wait_arrival(hops[2][0])
        fold_sub(2, 1)
        fold_sub(3, 0)
        finish_batch(0)
        wait_arrival(hops[2][1])
        fold_sub(3, 1)
        finish_batch(1)

        for rdma in started:
            rdma.wait_send()

    return pl.pallas_call(
        body,
        out_shape=jax.ShapeDtypeStruct((B, S_LOC, D), jnp.float32),
        in_specs=[pl.BlockSpec(memory_space=pltpu.VMEM)] * 7,
        out_specs=pl.BlockSpec(memory_space=pltpu.VMEM),
        scratch_shapes=[
            pltpu.VMEM((N_DEV, B * S_LOC, D), jnp.bfloat16),
            pltpu.VMEM((N_DEV, B * S_LOC, D), jnp.bfloat16),
            pltpu.VMEM((B * S_LOC, D), jnp.bfloat16),
            pltpu.VMEM((B * S_LOC, D), jnp.float32),
            pltpu.SemaphoreType.DMA((N_DEV - 1, B)),
            pltpu.SemaphoreType.DMA((N_DEV - 1, B)),
            pltpu.SemaphoreType.DMA((N_DEV - 1, B)),
            pltpu.SemaphoreType.DMA((N_DEV - 1, B)),
        ],
        compiler_params=_CompilerParams(
            collective_id=0, vmem_limit_bytes=100 * 1024 * 1024
        ),
    )(x_c, Wq_p, Wk_p, Wv_c, Wo_c, cos, sin)


# device time: 115525 ns/iter; 2.0575x vs baseline; 1.0535x over previous
import jax
import jax.numpy as jnp
import numpy as np
from jax import lax
from jax.experimental import pallas as pl
from jax.experimental.pallas import tpu as pltpu

N_DEV = 4
B = 2
S_LOC = 512
D = 1024
HQ = 8
DH = 128
HALF = D // 2
SCALE = 0.08838834764831843

_PERM = np.concatenate(
    [
        h * DH + np.concatenate([np.arange(0, DH, 2), np.arange(1, DH, 2)])
        for h in range(HQ)
    ]
)

_CompilerParams = getattr(pltpu, "CompilerParams", None) or getattr(
    pltpu, "TPUCompilerParams"
)


def kernel(x, Wq, Wk, Wv, Wo):
    my = lax.axis_index("i")

    pos = (my * S_LOC + jnp.arange(S_LOC))[:, None].astype(jnp.float32)
    inv = jnp.asarray(
        1.0 / (10000.0 ** (np.arange(0, DH, 2) / DH)), dtype=jnp.float32
    )
    ang = pos * inv[None, :]
    cos = jnp.concatenate([jnp.cos(ang), jnp.cos(ang)], axis=-1).astype(
        jnp.bfloat16
    )
    sin = jnp.concatenate([jnp.sin(ang), jnp.sin(ang)], axis=-1).astype(
        jnp.bfloat16
    )

    Wq_p = Wq[:, _PERM]
    Wk_p = Wk[:, _PERM]

    def body(
        x_ref,
        wq_ref,
        wk_ref,
        wv_ref,
        wo_ref,
        cos_ref,
        sin_ref,
        out_ref,
        rbuf,
        lbuf,
        acc_ref,
        r_send,
        r_recv,
        l_send,
        l_recv,
    ):
        my_pos = lax.axis_index("i")
        left = (my_pos - 1) % N_DEV
        right = (my_pos + 1) % N_DEV

        barrier_sem = pltpu.get_barrier_semaphore()
        for nbr in (left, right):
            pl.semaphore_signal(
                barrier_sem,
                inc=1,
                device_id=(nbr,),
                device_id_type=pl.DeviceIdType.MESH,
            )
        pl.semaphore_wait(barrier_sem, 2)

        cos_b = cos_ref[...]
        sin_b = sin_ref[...]
        wk_bf = wk_ref[...].astype(jnp.bfloat16)
        wv_bf = wv_ref[...].astype(jnp.bfloat16)

        def rope_batch(t):
            blocks = []
            for hd in range(HQ):
                tb = t[:, hd * DH : (hd + 1) * DH]
                tb_rot = jnp.concatenate(
                    [-tb[:, DH // 2 :], tb[:, : DH // 2]], axis=-1
                )
                blocks.append(tb * cos_b + tb_rot * sin_b)
            return jnp.concatenate(blocks, axis=-1)

        def proj_batch(b, w_bf, rope):
            xbb = x_ref[b, :, :].astype(jnp.bfloat16)
            t = jnp.dot(xbb, w_bf, preferred_element_type=jnp.float32)
            t = t.astype(jnp.bfloat16)
            return rope_batch(t) if rope else t

        started = []

        def rows(s):
            return pl.ds(s * S_LOC, S_LOC)

        def start_hop(h, s):
            r = pltpu.make_async_remote_copy(
                src_ref=rbuf.at[h, rows(s)],
                dst_ref=rbuf.at[h + 1, rows(s)],
                send_sem=r_send.at[h, s],
                recv_sem=r_recv.at[h, s],
                device_id=(right,),
                device_id_type=pl.DeviceIdType.MESH,
            )
            l = pltpu.make_async_remote_copy(
                src_ref=lbuf.at[h, rows(s)],
                dst_ref=lbuf.at[h + 1, rows(s)],
                send_sem=l_send.at[h, s],
                recv_sem=l_recv.at[h, s],
                device_id=(left,),
                device_id_type=pl.DeviceIdType.MESH,
            )
            r.start()
            l.start()
            started.append(r)
            started.append(l)
            return r, l

        def wait_arrival(pair):
            pair[0].wait_recv()
            pair[1].wait_recv()

        hops = {0: [None, None]}
        for s in range(B):
            kb = proj_batch(s, wk_bf, rope=True)
            vb = proj_batch(s, wv_bf, rope=False)
            rbuf[0, rows(s), :HALF] = kb[:, :HALF]
            rbuf[0, rows(s), HALF:] = vb[:, :HALF]
            lbuf[0, rows(s), :HALF] = kb[:, HALF:]
            lbuf[0, rows(s), HALF:] = vb[:, HALF:]
            hops[0][s] = start_hop(0, s)

        qscale = jnp.bfloat16(SCALE * 1.4426950408889634)
        wq_bf = wq_ref[...].astype(jnp.bfloat16)
        q_b = [(proj_batch(b, wq_bf, rope=True) * qscale) for b in range(B)]

        l_st = [[None] * HQ for _ in range(B)]

        def fold(b, hd, kbh, vbh, first):
            rs = slice(b * S_LOC, (b + 1) * S_LOC)
            cs = slice(hd * DH, (hd + 1) * DH)
            qbh = q_b[b][:, cs]
            s = lax.dot_general(
                qbh,
                kbh,
                (((1,), (1,)), ((), ())),
                preferred_element_type=jnp.float32,
            )
            p = jnp.exp2(s)
            pv = jnp.dot(
                p.astype(jnp.bfloat16), vbh, preferred_element_type=jnp.float32
            )
            lsum = jnp.sum(p, axis=-1, keepdims=True)
            if first:
                l_st[b][hd] = lsum
                acc_ref[rs, cs] = pv
            else:
                l_st[b][hd] = l_st[b][hd] + lsum
                acc_ref[rs, cs] = acc_ref[rs, cs] + pv

        def fold_sub(slot, b, first=False):
            rsl = rbuf[slot, b * S_LOC : (b + 1) * S_LOC, :]
            lsl = lbuf[slot, b * S_LOC : (b + 1) * S_LOC, :]
            for j in range(HQ // 2):
                cs = slice(j * DH, (j + 1) * DH)
                vs = slice(HALF + j * DH, HALF + (j + 1) * DH)
                fold(b, j, rsl[:, cs], rsl[:, vs], first)
                fold(b, 4 + j, lsl[:, cs], lsl[:, vs], first)

        def finish_batch(b, wo_bf):
            rs = pl.ds(b * S_LOC, S_LOC)
            for hd in range(HQ):
                cs = slice(hd * DH, (hd + 1) * DH)
                ctx = acc_ref[b * S_LOC : (b + 1) * S_LOC, cs] / l_st[b][hd]
                rbuf[0, rs, cs] = ctx.astype(jnp.bfloat16)
            out_b = jnp.dot(
                rbuf[0, rs, :], wo_bf, preferred_element_type=jnp.float32
            )
            out_ref[b, :, :] = out_b

        fold_sub(0, 0, first=True)
        wait_arrival(hops[0][0])
        hops[1] = [start_hop(1, 0), None]
        fold_sub(0, 1, first=True)
        wait_arrival(hops[0][1])
        hops[1][1] = start_hop(1, 1)
        fold_sub(1, 0)
        wait_arrival(hops[1][0])
        hops[2] = [start_hop(2, 0), None]
        fold_sub(1, 1)
        wait_arrival(hops[1][1])
        hops[2][1] = start_hop(2, 1)
        fold_sub(2, 0)
        wo_bf = wo_ref[...].astype(jnp.bfloat16)
        wait_arrival(hops[2][0])
        fold_sub(2, 1)
        fold_sub(3, 0)
        finish_batch(0, wo_bf)
        wait_arrival(hops[2][1])
        fold_sub(3, 1)
        finish_batch(1, wo_bf)

        for rdma in started:
            rdma.wait_send()

    return pl.pallas_call(
        body,
        out_shape=jax.ShapeDtypeStruct((B, S_LOC, D), jnp.float32),
        in_specs=[pl.BlockSpec(memory_space=pltpu.VMEM)] * 7,
        out_specs=pl.BlockSpec(memory_space=pltpu.VMEM),
        scratch_shapes=[
            pltpu.VMEM((N_DEV, B * S_LOC, D), jnp.bfloat16),
            pltpu.VMEM((N_DEV, B * S_LOC, D), jnp.bfloat16),
            pltpu.VMEM((B * S_LOC, D), jnp.float32),
            pltpu.SemaphoreType.DMA((N_DEV - 1, B)),
            pltpu.SemaphoreType.DMA((N_DEV - 1, B)),
            pltpu.SemaphoreType.DMA((N_DEV - 1, B)),
            pltpu.SemaphoreType.DMA((N_DEV - 1, B)),
        ],
        compiler_params=_CompilerParams(
            collective_id=0, vmem_limit_bytes=100 * 1024 * 1024
        ),
    )(x, Wq_p, Wk_p, Wv, Wo, cos, sin)
